# Initial kernel scaffold; baseline (speedup 1.0000x reference)
import jax
import jax.numpy as jnp
from jax.experimental import pallas as pl


def kernel(x, W1, b1, W2, b2, W3, b3, Wd, bd):
    raise NotImplementedError("write your pallas kernel here")


# speedup vs baseline: 5.1212x; 5.1212x over previous
import functools

import jax
import jax.numpy as jnp
from jax import lax
from jax.experimental import pallas as pl
from jax.experimental.pallas import tpu as pltpu
from jax.experimental.pallas import tpu_sc as plsc

_B, _N, _D, _K = 4, 2048, 64, 20
_BN_SCALE = (1.0 + 1e-3) ** -0.5

_ROWS = 256
_NBLK = (_B * _N) // _ROWS
_KPAD = 32

_NW = 32
_PTS_W = (_B * _N) // _NW
_CH_P = 16
_CH_R = _CH_P * _K
_NCH = _PTS_W // _CH_P
_IDXW = 64
_IDXROWS = 8


def _topk_body(x_ref, w1_ref, b1_ref, w2_ref, b2_ref, w3_ref, b3_ref,
               wd_ref, bd_ref, idx_ref):
    xb = x_ref[...]
    h = jnp.dot(xb, w1_ref[...], preferred_element_type=jnp.float32) + b1_ref[...]
    h = jnp.maximum(h, 0.0) * _BN_SCALE
    h = jnp.dot(h, w2_ref[...], preferred_element_type=jnp.float32) + b2_ref[...]
    h = jnp.maximum(h, 0.0) * _BN_SCALE
    h = jnp.dot(h, w3_ref[...], preferred_element_type=jnp.float32) + b3_ref[...]
    h = jnp.maximum(h, 0.0) * _BN_SCALE
    adj = jnp.dot(h, wd_ref[...], preferred_element_type=jnp.float32) + bd_ref[...]

    col = lax.broadcasted_iota(jnp.int32, (_ROWS, _N), 1)
    kcol = lax.broadcasted_iota(jnp.int32, (_ROWS, _KPAD), 1)
    base = pl.program_id(0) * _ROWS
    batch_off = (base // _N) * _N
    out = jnp.zeros((_ROWS, _KPAD), jnp.int32)
    for k in range(_K):
        m = jnp.min(adj, axis=1, keepdims=True)
        idx = jnp.min(jnp.where(adj == m, col, _N), axis=1, keepdims=True)
        adj = jnp.where(col == idx, jnp.inf, adj)
        out = jnp.where(kcol == k, idx + batch_off, out)
    idx_ref[...] = out


def _tc_topk(x_flat, w1, b1, w2, b2, w3, b3, wd, bd):
    full = lambda s: pl.BlockSpec(s, lambda i: (0, 0))
    return pl.pallas_call(
        _topk_body,
        grid=(_NBLK,),
        in_specs=[
            pl.BlockSpec((_ROWS, _D), lambda i: (i, 0)),
            full((_D, 32)), full((1, 32)),
            full((32, 64)), full((1, 64)),
            full((64, 512)), full((1, 512)),
            full((512, _N)), full((1, _N)),
        ],
        out_specs=pl.BlockSpec((_ROWS, _KPAD), lambda i: (i, 0)),
        out_shape=jax.ShapeDtypeStruct((_B * _N, _KPAD), jnp.int32),
    )(x_flat, w1, b1, w2, b2, w3, b3, wd, bd)


def _sc_edge_body(x_hbm, idx_hbm, out_hbm, idx_v, neigh_v, cent_v, out_v, sem):
    wid = lax.axis_index("s") * 2 + lax.axis_index("c")

    def chunk_body(ci, carry):
        p0 = wid * _PTS_W + ci * _CH_P
        r0 = p0 * _K
        chunk = wid * _NCH + ci
        pltpu.sync_copy(idx_hbm.at[pl.ds(chunk * _IDXROWS, _IDXROWS)], idx_v)
        copies = [
            pltpu.async_copy(x_hbm.at[idx_v.at[s]],
                             neigh_v.at[pl.ds(s * _IDXW, _IDXW)], sem)
            for s in range(_CH_R // _IDXW)
        ]
        pltpu.sync_copy(x_hbm.at[pl.ds(p0, _CH_P)], cent_v)
        for c in copies:
            c.wait()

        def pt_body(p, c2):
            for j in range(_D // 16):
                cvec = cent_v[p, pl.ds(j * 16, 16)]
                for k in range(_K):
                    r = p * _K + k
                    nvec = neigh_v[r, pl.ds(j * 16, 16)]
                    out_v[r, pl.ds(j * 16, 16)] = cvec
                    out_v[r, pl.ds(_D + j * 16, 16)] = nvec - cvec
            return c2

        lax.fori_loop(0, _CH_P, pt_body, 0)
        pltpu.sync_copy(out_v, out_hbm.at[pl.ds(r0, _CH_R)])
        return carry

    lax.fori_loop(0, _NCH, chunk_body, 0)


@functools.lru_cache(maxsize=1)
def _make_sc_edge():
    return functools.partial(
        pl.kernel,
        mesh=plsc.VectorSubcoreMesh(core_axis_name="c", subcore_axis_name="s"),
        out_type=jax.ShapeDtypeStruct((_B * _N * _K, 2 * _D), jnp.float32),
        scratch_types=[
            pltpu.VMEM((_IDXROWS, _IDXW), jnp.int32),
            pltpu.VMEM((_CH_R, 2 * _D), jnp.float32),
            pltpu.VMEM((_CH_P, 2 * _D), jnp.float32),
            pltpu.VMEM((_CH_R, 2 * _D), jnp.float32),
            pltpu.SemaphoreType.DMA,
        ],
    )(_sc_edge_body)


def kernel(x, W1, b1, W2, b2, W3, b3, Wd, bd):
    x_flat = x.reshape(_B * _N, _D)
    idx = _tc_topk(x_flat, W1, b1.reshape(1, -1), W2, b2.reshape(1, -1),
                   W3, b3.reshape(1, -1), Wd, bd.reshape(1, -1))
    nchunks = _NW * _NCH
    idx2d = jnp.pad(idx[:, :_K].reshape(nchunks, _CH_R), ((0, 0), (0, _IDXROWS * _IDXW - _CH_R))
                    ).reshape(nchunks * _IDXROWS, _IDXW)
    x_pad = jnp.pad(x_flat, ((0, 0), (0, _D)))
    out = _make_sc_edge()(x_pad, idx2d)
    return out.reshape(_B, _N, _K, 2 * _D)


# speedup vs baseline: 5.9046x; 1.1530x over previous
import functools

import jax
import jax.numpy as jnp
from jax import lax
from jax.experimental import pallas as pl
from jax.experimental.pallas import tpu as pltpu
from jax.experimental.pallas import tpu_sc as plsc

_B, _N, _D, _K = 4, 2048, 64, 20
_BN_SCALE = (1.0 + 1e-3) ** -0.5

_ROWS = 256
_NBLK = (_B * _N) // _ROWS
_KPAD = 32

_NW = 32
_PTS_W = (_B * _N) // _NW
_CH_P = 16
_CH_R = _CH_P * _K
_NCH = _PTS_W // _CH_P
_GR = 64


def _topk_body(x_ref, w1_ref, b1_ref, w2_ref, b2_ref, w3_ref, b3_ref,
               wd_ref, bd_ref, idx_ref, xpad_ref):
    xb = x_ref[...]
    h = jnp.dot(xb, w1_ref[...], preferred_element_type=jnp.float32) + b1_ref[...]
    h = jnp.maximum(h, 0.0) * _BN_SCALE
    h = jnp.dot(h, w2_ref[...], preferred_element_type=jnp.float32) + b2_ref[...]
    h = jnp.maximum(h, 0.0) * _BN_SCALE
    h = jnp.dot(h, w3_ref[...], preferred_element_type=jnp.float32) + b3_ref[...]
    h = jnp.maximum(h, 0.0) * _BN_SCALE
    adj = jnp.dot(h, wd_ref[...], preferred_element_type=jnp.float32) + bd_ref[...]

    xpad_ref[...] = jnp.concatenate([xb, jnp.zeros((_ROWS, _D), jnp.float32)], axis=1)

    col = lax.broadcasted_iota(jnp.int32, (_ROWS, _N), 1).astype(jnp.float32)
    kcol = lax.broadcasted_iota(jnp.int32, (_ROWS, _KPAD), 1).astype(jnp.float32)
    base = pl.program_id(0) * _ROWS
    batch_off = (base // _N) * _N
    out = jnp.zeros((_ROWS, _KPAD), jnp.float32)
    for k in range(_K):
        m = jnp.min(adj, axis=1, keepdims=True)
        idx = jnp.min(jnp.where(adj == m, col, float(_N)), axis=1, keepdims=True)
        adj = jnp.where(col == idx, jnp.inf, adj)
        out = jnp.where(kcol == float(k), idx, out)
    idx_ref[...] = out.astype(jnp.int32) + batch_off


def _tc_topk(x_flat, w1, b1, w2, b2, w3, b3, wd, bd):
    full = lambda s: pl.BlockSpec(s, lambda i: (0, 0))
    return pl.pallas_call(
        _topk_body,
        grid=(_NBLK,),
        in_specs=[
            pl.BlockSpec((_ROWS, _D), lambda i: (i, 0)),
            full((_D, 32)), full((1, 32)),
            full((32, 64)), full((1, 64)),
            full((64, 512)), full((1, 512)),
            full((512, _N)), full((1, _N)),
        ],
        out_specs=[
            pl.BlockSpec((_ROWS, _KPAD), lambda i: (i, 0)),
            pl.BlockSpec((_ROWS, 2 * _D), lambda i: (i, 0)),
        ],
        out_shape=[
            jax.ShapeDtypeStruct((_B * _N, _KPAD), jnp.int32),
            jax.ShapeDtypeStruct((_B * _N, 2 * _D), jnp.float32),
        ],
    )(x_flat, w1, b1, w2, b2, w3, b3, wd, bd)


def _sc_edge_body(x_hbm, idx_hbm, out_hbm, idxp_v, idx_v, neigh_v, cent_v,
                  out_v, sem):
    wid = lax.axis_index("s") * 2 + lax.axis_index("c")

    def chunk_body(ci, carry):
        p0 = wid * _PTS_W + ci * _CH_P
        r0 = p0 * _K
        pltpu.sync_copy(idx_hbm.at[pl.ds(p0, _CH_P)], idxp_v)
        for p in range(_CH_P):
            idx_v[pl.ds(20 * p, 16)] = idxp_v[p, pl.ds(0, 16)]
            idx_v[pl.ds(20 * p + 16, 16)] = idxp_v[p, pl.ds(16, 16)]
        copies = [
            pltpu.async_copy(x_hbm.at[idx_v.at[pl.ds(s * _GR, _GR)]],
                             neigh_v.at[pl.ds(s * _GR, _GR)], sem)
            for s in range(_CH_R // _GR)
        ]
        pltpu.sync_copy(x_hbm.at[pl.ds(p0, _CH_P)], cent_v)
        for c in copies:
            c.wait()

        def pt_body(p, c2):
            for j in range(_D // 16):
                cvec = cent_v[p, pl.ds(j * 16, 16)]
                for k in range(_K):
                    r = p * _K + k
                    nvec = neigh_v[r, pl.ds(j * 16, 16)]
                    out_v[r, pl.ds(j * 16, 16)] = cvec
                    out_v[r, pl.ds(_D + j * 16, 16)] = nvec - cvec
            return c2

        lax.fori_loop(0, _CH_P, pt_body, 0)
        pltpu.sync_copy(out_v, out_hbm.at[pl.ds(r0, _CH_R)])
        return carry

    lax.fori_loop(0, _NCH, chunk_body, 0)


@functools.lru_cache(maxsize=1)
def _make_sc_edge():
    return functools.partial(
        pl.kernel,
        mesh=plsc.VectorSubcoreMesh(core_axis_name="c", subcore_axis_name="s"),
        out_type=jax.ShapeDtypeStruct((_B * _N * _K, 2 * _D), jnp.float32),
        scratch_types=[
            pltpu.VMEM((_CH_P, _KPAD), jnp.int32),
            pltpu.VMEM((_CH_R + 16, ), jnp.int32),
            pltpu.VMEM((_CH_R, 2 * _D), jnp.float32),
            pltpu.VMEM((_CH_P, 2 * _D), jnp.float32),
            pltpu.VMEM((_CH_R, 2 * _D), jnp.float32),
            pltpu.SemaphoreType.DMA,
        ],
    )(_sc_edge_body)


def kernel(x, W1, b1, W2, b2, W3, b3, Wd, bd):
    x_flat = x.reshape(_B * _N, _D)
    idx, x_pad = _tc_topk(x_flat, W1, b1.reshape(1, -1), W2, b2.reshape(1, -1),
                          W3, b3.reshape(1, -1), Wd, bd.reshape(1, -1))
    out = _make_sc_edge()(x_pad, idx)
    return out.reshape(_B, _N, _K, 2 * _D)
